# Initial kernel scaffold; baseline (speedup 1.0000x reference)
import jax
import jax.numpy as jnp
from jax.experimental import pallas as pl


def kernel(x, importances_batch, W1, W2, fc_w, id_w, id_b):
    raise NotImplementedError("write your pallas kernel here")


# speedup vs baseline: 5.5124x; 5.5124x over previous
import jax
import jax.numpy as jnp
from jax.experimental import pallas as pl
from jax.experimental.pallas import tpu as pltpu
from functools import partial

_SIGMA = 2.0
_ADJ_RATIO = 0.2


def _kth_mask(S, k):
    K = S.shape[-1]
    bits = jax.lax.bitcast_convert_type(S, jnp.int32)

    def body(_, carry):
        lo, hi = carry
        mid = jax.lax.shift_right_logical(lo + hi, 1)
        cnt = jnp.sum((bits >= mid).astype(jnp.int32), axis=-1, keepdims=True)
        ge = cnt >= k
        return jnp.where(ge, mid, lo), jnp.where(ge, hi, mid)

    lo0 = jnp.zeros((K, 1), jnp.int32)
    hi0 = jnp.full((K, 1), 0x40000000, jnp.int32)
    lo, _ = jax.lax.fori_loop(0, 30, body, (lo0, hi0))
    return bits >= lo


def _sgcns_kernel(x_ref, imp_ref, w1_ref, w2_ref, fc_ref, idw_ref, idb_ref,
                  out_ref, emb1_ref, *, k_top):
    f32 = jnp.float32
    x = x_ref[0]
    imp = imp_ref[0]
    C, K = x.shape

    row = jax.lax.broadcasted_iota(jnp.int32, (K, K), 0)
    col = jax.lax.broadcasted_iota(jnp.int32, (K, K), 1)
    eye = (row == col).astype(f32)

    d = imp.reshape(K, 1) - imp
    Ap = jnp.exp(d * d * (-1.0 / (2.0 * _SIGMA * _SIGMA))) + eye
    inv_rp = 0.5 / (jnp.sum(Ap, axis=-1, keepdims=True) + 1e-8)

    def feature_adj(feat):
        ss = jnp.sum(feat * feat, axis=0, keepdims=True)
        inv_n = 1.0 / (jnp.sqrt(ss) + 1e-8)
        fn = feat * inv_n
        S = jax.lax.dot_general(fn, fn, (((0,), (0,)), ((), ())),
                                preferred_element_type=f32)
        S = jnp.maximum(S, 0.0)
        mask = _kth_mask(S, k_top)
        Af = jnp.where(mask, S, 0.0) + eye
        inv_rf = 0.5 / (jnp.sum(Af, axis=-1, keepdims=True) + 1e-8)
        return Af * inv_rf

    def layer(feat, W):
        A = Ap * inv_rp + feature_adj(feat)
        y = jax.lax.dot_general(W[...], feat, (((1,), (0,)), ((), ())),
                                preferred_element_type=f32)
        h = jax.lax.dot_general(y, A, (((1,), (0,)), ((), ())),
                                preferred_element_type=f32)
        return jnp.maximum(h, 0.0)

    emb1 = layer(x, w1_ref)
    emb1_ref[0] = emb1
    emb2 = layer(emb1, w2_ref)

    bag = jax.lax.dot_general(imp, emb2, (((1,), (1,)), ((), ())),
                              preferred_element_type=f32)
    xi = jax.lax.dot_general(imp, x, (((1,), (1,)), ((), ())),
                             preferred_element_type=f32)
    ident = jax.lax.dot_general(xi, idw_ref[...], (((1,), (1,)), ((), ())),
                                preferred_element_type=f32)
    ident = jnp.maximum(ident + idb_ref[...].reshape(1, -1), 0.0)
    out = jax.lax.dot_general(bag + ident, fc_ref[...], (((1,), (1,)), ((), ())),
                              preferred_element_type=f32)
    out_ref[0] = out


def kernel(x, importances_batch, W1, W2, fc_w, id_w, id_b):
    B, C, K = x.shape
    out_dim = W2.shape[0]
    class_num = fc_w.shape[0]
    k_top = max(int(_ADJ_RATIO * K), 1)

    rep = lambda *shape: pl.BlockSpec(shape, lambda b: (0,) * len(shape))
    out, emb1 = pl.pallas_call(
        partial(_sgcns_kernel, k_top=k_top),
        grid=(B,),
        in_specs=[
            pl.BlockSpec((1, C, K), lambda b: (b, 0, 0)),
            pl.BlockSpec((1, 1, K), lambda b: (b, 0, 0)),
            rep(*W1.shape),
            rep(*W2.shape),
            rep(*fc_w.shape),
            rep(*id_w.shape),
            rep(*id_b.shape),
        ],
        out_specs=[
            pl.BlockSpec((1, 1, class_num), lambda b: (b, 0, 0)),
            pl.BlockSpec((1, out_dim, K), lambda b: (b, 0, 0)),
        ],
        out_shape=[
            jax.ShapeDtypeStruct((B, 1, class_num), jnp.float32),
            jax.ShapeDtypeStruct((B, out_dim, K), jnp.float32),
        ],
        compiler_params=pltpu.CompilerParams(
            dimension_semantics=("arbitrary",),
            vmem_limit_bytes=100 * 1024 * 1024,
        ),
    )(x, importances_batch, W1, W2, fc_w, id_w, id_b)
    return (out.reshape(B, class_num), emb1)
